# baseline (speedup 1.0000x reference)
import functools

import jax
import jax.numpy as jnp
from jax import lax
from jax.experimental import pallas as pl
from jax.experimental.pallas import tpu as pltpu
from jax.experimental.pallas import tpu_sc as plsc

BATCH = 4096
D_GMF = 64
D_MLP = 256
NC = 2
NS = 16
NW = NC * NS
BPW = BATCH // NW
LANES = 16


def _sc_mlp_body(user_hbm, item_hbm, um_tbl, im_tbl, um_out, im_out,
                 idx_u, idx_i, em, emi, sem):
    wid = lax.axis_index("s") * NC + lax.axis_index("c")
    base = wid * BPW
    pltpu.sync_copy(user_hbm.at[pl.ds(base, BPW)], idx_u)
    pltpu.sync_copy(item_hbm.at[pl.ds(base, BPW)], idx_i)
    c1 = pltpu.async_copy(um_tbl.at[idx_u], em, sem)
    c2 = pltpu.async_copy(im_tbl.at[idx_i], emi, sem)
    c1.wait()
    pltpu.sync_copy(em, um_out.at[pl.ds(base, BPW)])
    c2.wait()
    pltpu.sync_copy(emi, im_out.at[pl.ds(base, BPW)])


def _sc_gmf_body(user_hbm, item_hbm, comb_tbl, gmf_out,
                 idx_u, idx_i, bu, bi, eg, sem):
    wid = lax.axis_index("s") * NC + lax.axis_index("c")
    base = wid * BPW
    pltpu.sync_copy(user_hbm.at[pl.ds(base, BPW)], idx_u)
    pltpu.sync_copy(item_hbm.at[pl.ds(base, BPW)], idx_i)
    c1 = pltpu.async_copy(comb_tbl.at[idx_u], bu, sem)
    c2 = pltpu.async_copy(comb_tbl.at[idx_i], bi, sem)
    c1.wait()
    c2.wait()

    def row(r, carry):
        for j in range(D_GMF // LANES):
            sl = pl.ds(j * LANES, LANES)
            eg[r, sl] = bu[r, sl] * bi[r, pl.ds(D_GMF + j * LANES, LANES)]
        return carry

    lax.fori_loop(0, BPW, row, 0)
    pltpu.sync_copy(eg, gmf_out.at[pl.ds(base, BPW)])


@functools.cache
def _make_sc_mlp_gather():
  return pl.kernel(
    _sc_mlp_body,
    out_type=[
        jax.ShapeDtypeStruct((BATCH, D_MLP), jnp.float32),
        jax.ShapeDtypeStruct((BATCH, D_MLP), jnp.float32),
    ],
    mesh=plsc.VectorSubcoreMesh(core_axis_name="c", subcore_axis_name="s"),
    scratch_types=[
        pltpu.VMEM((BPW,), jnp.int32),
        pltpu.VMEM((BPW,), jnp.int32),
        pltpu.VMEM((BPW, D_MLP), jnp.float32),
        pltpu.VMEM((BPW, D_MLP), jnp.float32),
        pltpu.SemaphoreType.DMA,
    ],
  )


@functools.cache
def _make_sc_gmf():
  return pl.kernel(
    _sc_gmf_body,
    out_type=[
        jax.ShapeDtypeStruct((BATCH, D_GMF), jnp.float32),
    ],
    mesh=plsc.VectorSubcoreMesh(core_axis_name="c", subcore_axis_name="s"),
    scratch_types=[
        pltpu.VMEM((BPW,), jnp.int32),
        pltpu.VMEM((BPW,), jnp.int32),
        pltpu.VMEM((BPW, 2 * D_GMF), jnp.float32),
        pltpu.VMEM((BPW, 2 * D_GMF), jnp.float32),
        pltpu.VMEM((BPW, D_GMF), jnp.float32),
        pltpu.SemaphoreType.DMA,
    ],
  )


TR_BLK = 16384


def _tr_body(ut_ref, it_ref, eye_ref, out_ref):
    dn = (((0,), (0,)), ((), ()))
    out_ref[:, :D_GMF] = lax.dot_general(
        ut_ref[...], eye_ref[...], dn, preferred_element_type=jnp.float32)
    out_ref[:, D_GMF:] = lax.dot_general(
        it_ref[...], eye_ref[...], dn, preferred_element_type=jnp.float32)


def _transpose_tables(ug_t, ig_t):
    n = ug_t.shape[1]
    grid = (n + TR_BLK - 1) // TR_BLK
    eye = jnp.eye(D_GMF, dtype=jnp.float32)
    return pl.pallas_call(
        _tr_body,
        grid=(grid,),
        in_specs=[
            pl.BlockSpec((D_GMF, TR_BLK), lambda i: (0, i)),
            pl.BlockSpec((D_GMF, TR_BLK), lambda i: (0, i)),
            pl.BlockSpec((D_GMF, D_GMF), lambda i: (0, 0)),
        ],
        out_specs=pl.BlockSpec((TR_BLK, 2 * D_GMF), lambda i: (i, 0)),
        out_shape=jax.ShapeDtypeStruct((n, 2 * D_GMF), jnp.float32),
    )(ug_t, ig_t, eye)


def _mlp_body(em_ref, emi_ref, w1a_ref, w1b_ref, w2_ref, w3_ref,
              b1_ref, b2_ref, b3_ref, out_ref):
    dn = (((1,), (1,)), ((), ()))
    h = lax.dot_general(em_ref[...], w1a_ref[...], dn,
                        preferred_element_type=jnp.float32)
    h += lax.dot_general(emi_ref[...], w1b_ref[...], dn,
                         preferred_element_type=jnp.float32)
    h = jnp.maximum(h + b1_ref[...], 0.0)
    h = lax.dot_general(h, w2_ref[...], dn, preferred_element_type=jnp.float32)
    h = jnp.maximum(h + b2_ref[...], 0.0)
    h = lax.dot_general(h, w3_ref[...], dn, preferred_element_type=jnp.float32)
    out_ref[...] = jnp.maximum(h + b3_ref[...], 0.0)


MLP_BLK = 1024


def _mlp(eu_mlp, ei_mlp, W1, b1, W2, b2, W3, b3):
    w1a = W1[:, :D_MLP]
    w1b = W1[:, D_MLP:]
    full = lambda shape: pl.BlockSpec(shape, lambda i: (0, 0))
    return pl.pallas_call(
        _mlp_body,
        grid=(BATCH // MLP_BLK,),
        in_specs=[
            pl.BlockSpec((MLP_BLK, D_MLP), lambda i: (i, 0)),
            pl.BlockSpec((MLP_BLK, D_MLP), lambda i: (i, 0)),
            full(w1a.shape), full(w1b.shape), full(W2.shape), full(W3.shape),
            full((1, 256)), full((1, 128)), full((1, 64)),
        ],
        out_specs=pl.BlockSpec((MLP_BLK, 64), lambda i: (i, 0)),
        out_shape=jax.ShapeDtypeStruct((BATCH, 64), jnp.float32),
    )(eu_mlp, ei_mlp, w1a, w1b, W2, W3,
      b1.reshape(1, -1), b2.reshape(1, -1), b3.reshape(1, -1))


def kernel(user, item, embed_user_GMF, embed_item_GMF,
           embed_user_MLP, embed_item_MLP, W1, b1, W2, b2, W3, b3):
    user = user.astype(jnp.int32)
    item = item.astype(jnp.int32)
    eu_mlp, ei_mlp = _make_sc_mlp_gather()(
        user, item, embed_user_MLP, embed_item_MLP)
    comb = _transpose_tables(embed_user_GMF.T, embed_item_GMF.T)
    (gmf,) = _make_sc_gmf()(user, item, comb)
    out_mlp = _mlp(eu_mlp, ei_mlp, W1, b1, W2, b2, W3, b3)
    return gmf, out_mlp


# speedup vs baseline: 1.0105x; 1.0105x over previous
import functools

import jax
import jax.numpy as jnp
from jax import lax
from jax.experimental import pallas as pl
from jax.experimental.pallas import tpu as pltpu
from jax.experimental.pallas import tpu_sc as plsc

BATCH = 4096
D_GMF = 64
D_MLP = 256
NC = 2
NS = 16
NW = NC * NS
BPW = BATCH // NW
LANES = 16


def _sc_mlp_body(user_hbm, item_hbm, um_tbl, im_tbl, um_out, im_out,
                 idx_u, idx_i, em, emi, sem):
    wid = lax.axis_index("s") * NC + lax.axis_index("c")
    base = wid * BPW
    pltpu.sync_copy(user_hbm.at[pl.ds(base, BPW)], idx_u)
    pltpu.sync_copy(item_hbm.at[pl.ds(base, BPW)], idx_i)
    c1 = pltpu.async_copy(um_tbl.at[idx_u], em, sem)
    c2 = pltpu.async_copy(im_tbl.at[idx_i], emi, sem)
    c1.wait()
    pltpu.sync_copy(em, um_out.at[pl.ds(base, BPW)])
    c2.wait()
    pltpu.sync_copy(emi, im_out.at[pl.ds(base, BPW)])


def _sc_gmf_body(user_hbm, item_hbm, comb_tbl, gmf_out,
                 idx_u, idx_i, bu, bi, eg, sem):
    wid = lax.axis_index("s") * NC + lax.axis_index("c")
    base = wid * BPW
    pltpu.sync_copy(user_hbm.at[pl.ds(base, BPW)], idx_u)
    pltpu.sync_copy(item_hbm.at[pl.ds(base, BPW)], idx_i)
    c1 = pltpu.async_copy(comb_tbl.at[idx_u], bu, sem)
    c2 = pltpu.async_copy(comb_tbl.at[idx_i], bi, sem)
    c1.wait()
    c2.wait()

    def row(r, carry):
        for j in range(D_GMF // LANES):
            sl = pl.ds(j * LANES, LANES)
            eg[r, sl] = bu[r, sl] * bi[r, pl.ds(D_GMF + j * LANES, LANES)]
        return carry

    lax.fori_loop(0, BPW, row, 0)
    pltpu.sync_copy(eg, gmf_out.at[pl.ds(base, BPW)])


@functools.cache
def _make_sc_mlp_gather():
  return pl.kernel(
    _sc_mlp_body,
    out_type=[
        jax.ShapeDtypeStruct((BATCH, D_MLP), jnp.float32),
        jax.ShapeDtypeStruct((BATCH, D_MLP), jnp.float32),
    ],
    mesh=plsc.VectorSubcoreMesh(core_axis_name="c", subcore_axis_name="s"),
    scratch_types=[
        pltpu.VMEM((BPW,), jnp.int32),
        pltpu.VMEM((BPW,), jnp.int32),
        pltpu.VMEM((BPW, D_MLP), jnp.float32),
        pltpu.VMEM((BPW, D_MLP), jnp.float32),
        pltpu.SemaphoreType.DMA,
    ],
  )


@functools.cache
def _make_sc_gmf():
  return pl.kernel(
    _sc_gmf_body,
    out_type=[
        jax.ShapeDtypeStruct((BATCH, D_GMF), jnp.float32),
    ],
    mesh=plsc.VectorSubcoreMesh(core_axis_name="c", subcore_axis_name="s"),
    scratch_types=[
        pltpu.VMEM((BPW,), jnp.int32),
        pltpu.VMEM((BPW,), jnp.int32),
        pltpu.VMEM((BPW, 2 * D_GMF), jnp.float32),
        pltpu.VMEM((BPW, 2 * D_GMF), jnp.float32),
        pltpu.VMEM((BPW, D_GMF), jnp.float32),
        pltpu.SemaphoreType.DMA,
    ],
  )


TR_BLK = 16384


def _tr_body(ut_ref, it_ref, out_ref):
    out_ref[...] = jnp.concatenate((ut_ref[...].T, it_ref[...].T), axis=1)


def _transpose_tables(ug_t, ig_t):
    n = ug_t.shape[1]
    grid = (n + TR_BLK - 1) // TR_BLK
    return pl.pallas_call(
        _tr_body,
        grid=(grid,),
        in_specs=[
            pl.BlockSpec((D_GMF, TR_BLK), lambda i: (0, i)),
            pl.BlockSpec((D_GMF, TR_BLK), lambda i: (0, i)),
        ],
        out_specs=pl.BlockSpec((TR_BLK, 2 * D_GMF), lambda i: (i, 0)),
        out_shape=jax.ShapeDtypeStruct((n, 2 * D_GMF), jnp.float32),
    )(ug_t, ig_t)


def _mlp_body(em_ref, emi_ref, w1a_ref, w1b_ref, w2_ref, w3_ref,
              b1_ref, b2_ref, b3_ref, out_ref):
    dn = (((1,), (1,)), ((), ()))
    h = lax.dot_general(em_ref[...], w1a_ref[...], dn,
                        preferred_element_type=jnp.float32)
    h += lax.dot_general(emi_ref[...], w1b_ref[...], dn,
                         preferred_element_type=jnp.float32)
    h = jnp.maximum(h + b1_ref[...], 0.0)
    h = lax.dot_general(h, w2_ref[...], dn, preferred_element_type=jnp.float32)
    h = jnp.maximum(h + b2_ref[...], 0.0)
    h = lax.dot_general(h, w3_ref[...], dn, preferred_element_type=jnp.float32)
    out_ref[...] = jnp.maximum(h + b3_ref[...], 0.0)


MLP_BLK = 2048


def _mlp(eu_mlp, ei_mlp, W1, b1, W2, b2, W3, b3):
    w1a = W1[:, :D_MLP]
    w1b = W1[:, D_MLP:]
    full = lambda shape: pl.BlockSpec(shape, lambda i: (0, 0))
    return pl.pallas_call(
        _mlp_body,
        grid=(BATCH // MLP_BLK,),
        in_specs=[
            pl.BlockSpec((MLP_BLK, D_MLP), lambda i: (i, 0)),
            pl.BlockSpec((MLP_BLK, D_MLP), lambda i: (i, 0)),
            full(w1a.shape), full(w1b.shape), full(W2.shape), full(W3.shape),
            full((1, 256)), full((1, 128)), full((1, 64)),
        ],
        out_specs=pl.BlockSpec((MLP_BLK, 64), lambda i: (i, 0)),
        out_shape=jax.ShapeDtypeStruct((BATCH, 64), jnp.float32),
    )(eu_mlp, ei_mlp, w1a, w1b, W2, W3,
      b1.reshape(1, -1), b2.reshape(1, -1), b3.reshape(1, -1))


def kernel(user, item, embed_user_GMF, embed_item_GMF,
           embed_user_MLP, embed_item_MLP, W1, b1, W2, b2, W3, b3):
    user = user.astype(jnp.int32)
    item = item.astype(jnp.int32)
    eu_mlp, ei_mlp = _make_sc_mlp_gather()(
        user, item, embed_user_MLP, embed_item_MLP)
    comb = _transpose_tables(embed_user_GMF.T, embed_item_GMF.T)
    (gmf,) = _make_sc_gmf()(user, item, comb)
    out_mlp = _mlp(eu_mlp, ei_mlp, W1, b1, W2, b2, W3, b3)
    return gmf, out_mlp


# speedup vs baseline: 1.0543x; 1.0433x over previous
import functools

import jax
import jax.numpy as jnp
from jax import lax
from jax.experimental import pallas as pl
from jax.experimental.pallas import tpu as pltpu
from jax.experimental.pallas import tpu_sc as plsc

BATCH = 4096
D_GMF = 64
D_MLP = 256
NC = 2
NS = 16
NW = NC * NS
BPW = BATCH // NW
LANES = 16


def _sc_mlp_body(user_hbm, item_hbm, um_tbl, im_tbl, um_out, im_out,
                 idx_u, idx_i, em, emi, sem):
    wid = lax.axis_index("s") * NC + lax.axis_index("c")
    base = wid * BPW
    pltpu.sync_copy(user_hbm.at[pl.ds(base, BPW)], idx_u)
    pltpu.sync_copy(item_hbm.at[pl.ds(base, BPW)], idx_i)
    c1 = pltpu.async_copy(um_tbl.at[idx_u], em, sem)
    c2 = pltpu.async_copy(im_tbl.at[idx_i], emi, sem)
    c1.wait()
    pltpu.sync_copy(em, um_out.at[pl.ds(base, BPW)])
    c2.wait()
    pltpu.sync_copy(emi, im_out.at[pl.ds(base, BPW)])


def _sc_gmf_body(user_hbm, item_hbm, comb_tbl, gmf_out,
                 idx_u, idx_i, bu, bi, eg, sem):
    wid = lax.axis_index("s") * NC + lax.axis_index("c")
    base = wid * BPW
    pltpu.sync_copy(user_hbm.at[pl.ds(base, BPW)], idx_u)
    pltpu.sync_copy(item_hbm.at[pl.ds(base, BPW)], idx_i)
    c1 = pltpu.async_copy(comb_tbl.at[idx_u], bu, sem)
    c2 = pltpu.async_copy(comb_tbl.at[idx_i], bi, sem)
    c1.wait()
    c2.wait()

    def row(r, carry):
        for j in range(D_GMF // LANES):
            sl = pl.ds(j * LANES, LANES)
            eg[r, sl] = bu[r, sl] * bi[r, pl.ds(D_GMF + j * LANES, LANES)]
        return carry

    lax.fori_loop(0, BPW, row, 0)
    pltpu.sync_copy(eg, gmf_out.at[pl.ds(base, BPW)])


@functools.cache
def _make_sc_mlp_gather():
  return pl.kernel(
    _sc_mlp_body,
    out_type=[
        jax.ShapeDtypeStruct((BATCH, D_MLP), jnp.float32),
        jax.ShapeDtypeStruct((BATCH, D_MLP), jnp.float32),
    ],
    mesh=plsc.VectorSubcoreMesh(core_axis_name="c", subcore_axis_name="s"),
    scratch_types=[
        pltpu.VMEM((BPW,), jnp.int32),
        pltpu.VMEM((BPW,), jnp.int32),
        pltpu.VMEM((BPW, D_MLP), jnp.float32),
        pltpu.VMEM((BPW, D_MLP), jnp.float32),
        pltpu.SemaphoreType.DMA,
    ],
  )


@functools.cache
def _make_sc_gmf():
  return pl.kernel(
    _sc_gmf_body,
    out_type=[
        jax.ShapeDtypeStruct((BATCH, D_GMF), jnp.float32),
    ],
    mesh=plsc.VectorSubcoreMesh(core_axis_name="c", subcore_axis_name="s"),
    scratch_types=[
        pltpu.VMEM((BPW,), jnp.int32),
        pltpu.VMEM((BPW,), jnp.int32),
        pltpu.VMEM((BPW, 2 * D_GMF), jnp.float32),
        pltpu.VMEM((BPW, 2 * D_GMF), jnp.float32),
        pltpu.VMEM((BPW, D_GMF), jnp.float32),
        pltpu.SemaphoreType.DMA,
    ],
  )


TR_BLK = 16384


def _tr_body(ut_ref, it_ref, out_ref):
    out_ref[...] = jnp.concatenate((ut_ref[...].T, it_ref[...].T), axis=1)


def _transpose_tables(ug_t, ig_t):
    n = ug_t.shape[1]
    grid = (n + TR_BLK - 1) // TR_BLK
    return pl.pallas_call(
        _tr_body,
        grid=(grid,),
        in_specs=[
            pl.BlockSpec((D_GMF, TR_BLK), lambda i: (0, i)),
            pl.BlockSpec((D_GMF, TR_BLK), lambda i: (0, i)),
        ],
        out_specs=pl.BlockSpec((TR_BLK, 2 * D_GMF), lambda i: (i, 0)),
        out_shape=jax.ShapeDtypeStruct((n, 2 * D_GMF), jnp.float32),
    )(ug_t, ig_t)


def _mlp_body(em_ref, emi_ref, w1a_ref, w1b_ref, w2_ref, w3_ref,
              b1_ref, b2_ref, b3_ref, out_ref):
    dn = (((1,), (1,)), ((), ()))
    h = lax.dot_general(em_ref[...], w1a_ref[...], dn,
                        preferred_element_type=jnp.float32)
    h += lax.dot_general(emi_ref[...], w1b_ref[...], dn,
                         preferred_element_type=jnp.float32)
    h = jnp.maximum(h + b1_ref[...], 0.0)
    h = lax.dot_general(h, w2_ref[...], dn, preferred_element_type=jnp.float32)
    h = jnp.maximum(h + b2_ref[...], 0.0)
    h = lax.dot_general(h, w3_ref[...], dn, preferred_element_type=jnp.float32)
    out_ref[...] = jnp.maximum(h + b3_ref[...], 0.0).T


MLP_BLK = 2048


def _mlp(eu_mlp, ei_mlp, W1, b1, W2, b2, W3, b3):
    w1a = W1[:, :D_MLP]
    w1b = W1[:, D_MLP:]
    full = lambda shape: pl.BlockSpec(shape, lambda i: (0, 0))
    return pl.pallas_call(
        _mlp_body,
        grid=(BATCH // MLP_BLK,),
        in_specs=[
            pl.BlockSpec((MLP_BLK, D_MLP), lambda i: (i, 0)),
            pl.BlockSpec((MLP_BLK, D_MLP), lambda i: (i, 0)),
            full(w1a.shape), full(w1b.shape), full(W2.shape), full(W3.shape),
            full((1, 256)), full((1, 128)), full((1, 64)),
        ],
        out_specs=pl.BlockSpec((64, MLP_BLK), lambda i: (0, i)),
        out_shape=jax.ShapeDtypeStruct((64, BATCH), jnp.float32),
    )(eu_mlp, ei_mlp, w1a, w1b, W2, W3,
      b1.reshape(1, -1), b2.reshape(1, -1), b3.reshape(1, -1)).T


def kernel(user, item, embed_user_GMF, embed_item_GMF,
           embed_user_MLP, embed_item_MLP, W1, b1, W2, b2, W3, b3):
    user = user.astype(jnp.int32)
    item = item.astype(jnp.int32)
    eu_mlp, ei_mlp = _make_sc_mlp_gather()(
        user, item, embed_user_MLP, embed_item_MLP)
    comb = _transpose_tables(embed_user_GMF.T, embed_item_GMF.T)
    (gmf,) = _make_sc_gmf()(user, item, comb)
    out_mlp = _mlp(eu_mlp, ei_mlp, W1, b1, W2, b2, W3, b3)
    return gmf, out_mlp
